# baseline (speedup 1.0000x reference)
import functools

import jax
import jax.numpy as jnp
from jax import lax
from jax.experimental import pallas as pl
from jax.experimental.pallas import tpu as pltpu
from jax.experimental.pallas import tpu_sc as plsc

N = 10000
E = 160000
DIN = 256
D = 512
FF = 1024
L = 2

RB = 1000
NCH = 4
CW = 128

NW = 32
EPAD = 163840
EW = EPAD // NW
B1 = 32
NB1 = EW // B1
B3 = 128
NB3 = EW // B3
NPAD = 12288
NR = 3
RSPAN = 4736
NSPM = RSPAN + 8
NSLICE = RSPAN // 16

_MESH = plsc.VectorSubcoreMesh(core_axis_name="c", subcore_axis_name="s")
_SC_PARAMS = pltpu.CompilerParams(needs_layout_passes=False)


def _relu6(v):
    return jnp.clip(v, 0.0, 6.0)


def _full(*shape):
    return pl.BlockSpec(shape, lambda i: (0,) * len(shape))


def _mlp_body(x_ref, w0_ref, b0_ref, w1_ref, b1_ref, w2_ref, b2_ref, out_ref):
    h = _relu6(jnp.dot(x_ref[...], w0_ref[...], preferred_element_type=jnp.float32) + b0_ref[...])
    h = _relu6(jnp.dot(h, w1_ref[...], preferred_element_type=jnp.float32) + b1_ref[...])
    out_ref[...] = jnp.dot(h, w2_ref[...], preferred_element_type=jnp.float32) + b2_ref[...]


def _mlp(x, mW0, mb0, mW1, mb1, mW2, mb2):
    return pl.pallas_call(
        _mlp_body,
        grid=(N // RB,),
        in_specs=[
            pl.BlockSpec((RB, DIN), lambda i: (i, 0)),
            _full(DIN, 512), _full(1, 512), _full(512, 256), _full(1, 256),
            _full(256, D), _full(1, D),
        ],
        out_specs=pl.BlockSpec((RB, D), lambda i: (i, 0)),
        out_shape=jax.ShapeDtypeStruct((N, D), jnp.float32),
    )(x, mW0, mb0.reshape(1, 512), mW1, mb1.reshape(1, 256), mW2, mb2.reshape(1, D))


def _pre_body(h_ref, wl_ref, bl_ref, wr_ref, br_ref, wc_ref, bc_ref,
              xl4_ref, xr4_ref, li_ref):
    h = h_ref[...]
    xl = jnp.dot(h, wl_ref[...], preferred_element_type=jnp.float32) + bl_ref[...]
    xr = jnp.dot(h, wr_ref[...], preferred_element_type=jnp.float32) + br_ref[...]
    for c in range(NCH):
        xl4_ref[c] = xl[:, c * CW:(c + 1) * CW]
        xr4_ref[c] = xr[:, c * CW:(c + 1) * CW]
    li_ref[...] = jnp.dot(h, wc_ref[...], preferred_element_type=jnp.float32) + bc_ref[...]


def _pre(h, gWl, gbl, gWr, gbr, lW, lb):
    chunk4 = pl.BlockSpec((NCH, RB, CW), lambda i: (0, i, 0))
    return pl.pallas_call(
        _pre_body,
        grid=(N // RB,),
        in_specs=[
            pl.BlockSpec((RB, D), lambda i: (i, 0)),
            _full(D, D), _full(1, D), _full(D, D), _full(1, D), _full(D, D), _full(1, D),
        ],
        out_specs=[chunk4, chunk4, pl.BlockSpec((RB, D), lambda i: (i, 0))],
        out_shape=[jax.ShapeDtypeStruct((NCH, N, CW), jnp.float32)] * 2
        + [jax.ShapeDtypeStruct((N, D), jnp.float32)],
    )(h, gWl, gbl.reshape(1, D), gWr, gbr.reshape(1, D), lW, lb.reshape(1, D))


def _post_body(sp2_ref, li_ref, h_ref, cwt_ref, cwb_ref, cbe_ref,
               w1_ref, b1_ref, w2_ref, b2_ref, out_ref):
    acc = cbe_ref[...] + h_ref[...]
    for p in range(2):
        for c in range(NCH):
            acc = acc + jnp.dot(sp2_ref[p, c], cwt_ref[c], preferred_element_type=jnp.float32)
    fus = acc + jnp.dot(li_ref[...], cwb_ref[...], preferred_element_type=jnp.float32)
    ffn = _relu6(jnp.dot(fus, w1_ref[...], preferred_element_type=jnp.float32) + b1_ref[...])
    ffn = jnp.dot(ffn, w2_ref[...], preferred_element_type=jnp.float32) + b2_ref[...]
    out_ref[...] = _relu6(fus + ffn)


def _post(sp2, li, h, cW, cb_eff, fW1, fb1, fW2, fb2):
    cwt = cW[:D].reshape(NCH, CW, D)
    cwb = cW[D:]
    return pl.pallas_call(
        _post_body,
        grid=(N // RB,),
        in_specs=[
            pl.BlockSpec((2, NCH, RB, CW), lambda i: (0, 0, i, 0)),
            pl.BlockSpec((RB, D), lambda i: (i, 0)),
            pl.BlockSpec((RB, D), lambda i: (i, 0)),
            _full(NCH, CW, D), _full(D, D), _full(1, D),
            _full(D, FF), _full(1, FF), _full(FF, D), _full(1, D),
        ],
        out_specs=pl.BlockSpec((RB, D), lambda i: (i, 0)),
        out_shape=jax.ShapeDtypeStruct((N, D), jnp.float32),
    )(sp2, li, h, cwt, cwb, cb_eff.reshape(1, D),
      fW1, fb1.reshape(1, FF), fW2, fb2.reshape(1, D))


def _wid():
    return lax.axis_index("s") * 2 + lax.axis_index("c")


def _make_sc_alpha():
    @functools.partial(
        pl.kernel,
        out_type=[
            jax.ShapeDtypeStruct((EPAD,), jnp.float32),
            jax.ShapeDtypeStruct((NW * 128,), jnp.float32),
        ],
        mesh=_MESH,
        compiler_params=_SC_PARAMS,
        scratch_types=[
            pltpu.VMEM((EW,), jnp.int32),
            pltpu.VMEM((EW,), jnp.int32),
            pltpu.VMEM((EW,), jnp.float32),
            pltpu.VMEM((D,), jnp.float32),
            pltpu.VMEM((D,), jnp.float32),
            pltpu.VMEM((2, NCH, B1, CW), jnp.float32),
            pltpu.VMEM((2, NCH, B1, CW), jnp.float32),
            pltpu.VMEM((256,), jnp.float32),
            pltpu.VMEM((EW,), jnp.float32),
            pltpu.VMEM((128,), jnp.float32),
            pltpu.SemaphoreType.DMA,
            pltpu.SemaphoreType.DMA,
        ],
    )
    def sc_alpha(xl4, xr4,
                 src_hbm, dst_hbm, ea_hbm, gwe_hbm, gatt_hbm,
                 alpha_hbm, wmax_hbm,
                 src_v, dst_v, ea_v, gwe_v, gatt_v, bufl, bufr, partials,
                 alpha_v, wmax_v, sem0, sem1):
        xlc = tuple(xl4.at[c] for c in range(NCH))
        xrc = tuple(xr4.at[c] for c in range(NCH))
        w = _wid()
        base = w * EW
        pltpu.sync_copy(src_hbm.at[pl.ds(base, EW)], src_v)
        pltpu.sync_copy(dst_hbm.at[pl.ds(base, EW)], dst_v)
        pltpu.sync_copy(ea_hbm.at[pl.ds(base, EW)], ea_v)
        pltpu.sync_copy(gwe_hbm, gwe_v)
        pltpu.sync_copy(gatt_hbm, gatt_v)

        sems = (sem0, sem1)

        def issue(b, par):
            sidx = src_v.at[pl.ds(b * B1, B1)]
            didx = dst_v.at[pl.ds(b * B1, B1)]
            for c in range(NCH):
                pltpu.async_copy(xlc[c].at[sidx], bufl.at[par].at[c], sems[par])
                pltpu.async_copy(xrc[c].at[didx], bufr.at[par].at[c], sems[par])

        def wait_all(par):
            for c in range(NCH):
                pltpu.make_async_copy(xlc[c].at[pl.ds(0, B1)],
                                      bufl.at[par].at[c], sems[par]).wait()
                pltpu.make_async_copy(xrc[c].at[pl.ds(0, B1)],
                                      bufr.at[par].at[c], sems[par]).wait()

        def compute(b, par, maxvec):
            def group(g, mv):
                ebase = b * B1 + g * 16
                ea16 = ea_v[pl.ds(ebase, 16)]
                eab = [ea16[j] for j in range(16)]
                acc = [jnp.zeros((16,), jnp.float32) for _ in range(16)]
                for k in range(D // 16):
                    c, kk = divmod(k, CW // 16)
                    gwe_k = gwe_v[pl.ds(k * 16, 16)]
                    gatt_k = gatt_v[pl.ds(k * 16, 16)]
                    for j in range(16):
                        t = (bufl[par, c, g * 16 + j, pl.ds(kk * 16, 16)]
                             + bufr[par, c, g * 16 + j, pl.ds(kk * 16, 16)]
                             + eab[j] * gwe_k)
                        t = jnp.maximum(t, 0.2 * t)
                        acc[j] = acc[j] + t * gatt_k
                for j in range(16):
                    partials[pl.ds(j * 16, 16)] = acc[j]
                iota16 = lax.iota(jnp.int32, 16) * 16
                tot = jnp.zeros((16,), jnp.float32)
                for t_ in range(16):
                    col = plsc.load_gather(partials, [iota16 + t_])
                    tot = tot + col
                alpha_v[pl.ds(ebase, 16)] = tot
                return jnp.maximum(mv, tot)

            return lax.fori_loop(0, B1 // 16, group, maxvec)

        issue(0, 0)
        issue(1, 1)

        def super2(s2, maxvec):
            b0 = s2 * 2
            wait_all(0)
            maxvec = compute(b0, 0, maxvec)

            @pl.when(b0 + 2 < NB1)
            def _i0():
                issue(b0 + 2, 0)

            wait_all(1)
            maxvec = compute(b0 + 1, 1, maxvec)

            @pl.when(b0 + 3 < NB1)
            def _i1():
                issue(b0 + 3, 1)

            return maxvec

        maxvec = lax.fori_loop(0, NB1 // 2, super2,
                               jnp.full((16,), -3.4e38, jnp.float32))
        wmax_v[pl.ds(0, 16)] = maxvec
        for u in range(1, 8):
            wmax_v[pl.ds(u * 16, 16)] = jnp.full((16,), -3.4e38, jnp.float32)
        pltpu.sync_copy(alpha_v, alpha_hbm.at[pl.ds(base, EW)])
        pltpu.sync_copy(wmax_v, wmax_hbm.at[pl.ds(w * 128, 128)])

    return sc_alpha


def _make_sc_softmax():
    @functools.partial(
        pl.kernel,
        out_type=[
            jax.ShapeDtypeStruct((EPAD,), jnp.float32),
            jax.ShapeDtypeStruct((NW, NPAD), jnp.float32),
        ],
        mesh=_MESH,
        compiler_params=_SC_PARAMS,
        scratch_types=[
            pltpu.VMEM((EW,), jnp.float32),
            pltpu.VMEM((EW,), jnp.int32),
            pltpu.VMEM((EW,), jnp.float32),
            pltpu.VMEM((NW * 128,), jnp.float32),
            pltpu.VMEM((NPAD,), jnp.float32),
        ],
    )
    def sc_softmax(alpha_hbm, wmax_hbm, dst_hbm, ae_hbm, asums_hbm,
                   alpha_v, dst_v, ae_v, wmax_v, asum_v):
        w = _wid()
        base = w * EW
        pltpu.sync_copy(alpha_hbm.at[pl.ds(base, EW)], alpha_v)
        pltpu.sync_copy(dst_hbm.at[pl.ds(base, EW)], dst_v)
        pltpu.sync_copy(wmax_hbm, wmax_v)
        gmax = wmax_v[pl.ds(0, 16)]
        for k in range(1, NW * 8):
            gmax = jnp.maximum(gmax, wmax_v[pl.ds(k * 16, 16)])
        cmax = jnp.max(gmax)

        def zero(i, _):
            for u in range(8):
                asum_v[pl.ds(i * 128 + u * 16, 16)] = jnp.zeros((16,), jnp.float32)
            return 0

        lax.fori_loop(0, NPAD // 128, zero, 0)
        iota = lax.iota(jnp.int32, 16)

        def group(g, _):
            ebase = g * 16
            alpha16 = alpha_v[pl.ds(ebase, 16)]
            ae16 = jnp.exp(alpha16 - cmax)
            valid = (base + ebase + iota) < E
            ae16 = jnp.where(valid, ae16, 0.0)
            ae_v[pl.ds(ebase, 16)] = ae16
            dst16 = dst_v[pl.ds(ebase, 16)]
            plsc.addupdate_scatter(asum_v, [dst16], ae16)
            return 0

        lax.fori_loop(0, EW // 16, group, 0)
        pltpu.sync_copy(ae_v, ae_hbm.at[pl.ds(base, EW)])
        pltpu.sync_copy(asum_v, asums_hbm.at[w])

    return sc_softmax


def _make_sc_combine():
    @functools.partial(
        pl.kernel,
        out_type=jax.ShapeDtypeStruct((NPAD,), jnp.float32),
        mesh=_MESH,
        compiler_params=_SC_PARAMS,
        scratch_types=[
            pltpu.VMEM((384,), jnp.float32),
            pltpu.VMEM((384,), jnp.float32),
        ],
    )
    def sc_combine(asums_hbm, asum_hbm, acc_v, tmp_v):
        w = _wid()
        for t in range(24):
            acc_v[pl.ds(t * 16, 16)] = jnp.zeros((16,), jnp.float32)
        for k in range(NW):
            pltpu.sync_copy(asums_hbm.at[k].at[pl.ds(w * 384, 384)], tmp_v)
            for t in range(24):
                sl = pl.ds(t * 16, 16)
                acc_v[sl] = acc_v[sl] + tmp_v[sl]
        pltpu.sync_copy(acc_v, asum_hbm.at[pl.ds(w * 384, 384)])

    return sc_combine


def _make_sc_sp():
    @functools.partial(
        pl.kernel,
        out_type=jax.ShapeDtypeStruct((2, NCH, NR * RSPAN, CW), jnp.float32),
        mesh=_MESH,
        compiler_params=_SC_PARAMS,
        scratch_types=[
            pltpu.VMEM((EW,), jnp.int32),
            pltpu.VMEM((NB3, B3), jnp.int32),
            pltpu.VMEM((NR, NB3, B3), jnp.int32),
            pltpu.VMEM((EW,), jnp.float32),
            pltpu.VMEM((NPAD,), jnp.float32),
            pltpu.VMEM((EW,), jnp.float32),
            pltpu.VMEM((2, B3, CW), jnp.float32),
            pltpu.VMEM((64, CW), jnp.float32),
            pltpu.VMEM_SHARED((NSPM, CW), jnp.float32),
            pltpu.SemaphoreType.DMA,
        ],
    )
    def sc_sp(xl4,
              src_hbm, dst2_hbm, ae_hbm, asum_hbm, sp_hbm,
              src_v, dst2_v, dstR_v, ae_v, asum_v, a_v, rows, zbuf,
              spmem, semg):
        core = lax.axis_index("c")
        sub = lax.axis_index("s")
        w = sub * 2 + core
        base = w * EW
        pltpu.sync_copy(src_hbm.at[pl.ds(base, EW)], src_v)
        pltpu.sync_copy(dst2_hbm.at[w], dst2_v)
        pltpu.sync_copy(ae_hbm.at[pl.ds(base, EW)], ae_v)
        pltpu.sync_copy(asum_hbm, asum_v)

        def agroup(g, _):
            ebase = g * 16
            ae16 = ae_v[pl.ds(ebase, 16)]
            r_ = g // 8
            cc = g % 8
            dst16 = dst2_v[r_, pl.ds(cc * 16, 16)]
            as16 = plsc.load_gather(asum_v, [dst16])
            a_v[pl.ds(ebase, 16)] = ae16 / (as16 + 1e-16)
            trash = jnp.full((16,), RSPAN, jnp.int32)
            for rr in range(NR):
                dd = dst16 - rr * RSPAN
                ok = jnp.logical_and(dd >= 0, dd < RSPAN)
                dstR_v[rr, r_, pl.ds(cc * 16, 16)] = jnp.where(ok, dd, trash)
            return 0

        lax.fori_loop(0, EW // 16, agroup, 0)

        def zrow(i, _):
            for u in range(CW // 16):
                zbuf[i, pl.ds(u * 16, 16)] = jnp.zeros((16,), jnp.float32)
            return 0

        lax.fori_loop(0, 64, zrow, 0)

        def rbody(r, _r):
            dref = dstR_v.at[r]

            def cbody(c, _c):
                for z in range(4):
                    pltpu.sync_copy(
                        zbuf, spmem.at[pl.ds(sub * NSLICE + z * 64, 64)])
                pltpu.sync_copy(
                    zbuf.at[pl.ds(0, 40)],
                    spmem.at[pl.ds(sub * NSLICE + 256, 40)])

                @pl.when(sub == 0)
                def _zero_trash():
                    pltpu.sync_copy(zbuf, spmem.at[pl.ds(RSPAN - 56, 64)])
                plsc.subcore_barrier()

                def issueg(b, par):
                    sidx = src_v.at[pl.ds(b * B3, B3)]
                    pltpu.async_copy(xl4.at[c].at[sidx], rows.at[par], semg)

                def waitg(par):
                    pltpu.make_async_copy(xl4.at[c].at[pl.ds(0, B3)],
                                          rows.at[par], semg).wait()

                def scale(b, par):
                    def sgroup(g, _g):
                        ebase = b * B3 + g * 16
                        a16 = a_v[pl.ds(ebase, 16)]
                        for j in range(16):
                            aj = a16[j]
                            for k in range(CW // 16):
                                sl = pl.ds(k * 16, 16)
                                rows[par, g * 16 + j, sl] = rows[par, g * 16 + j, sl] * aj
                        return 0

                    lax.fori_loop(0, B3 // 16, sgroup, 0)

                issueg(0, 0)
                issueg(1, 1)

                def super2(s2, _):
                    b0 = s2 * 2
                    waitg(0)
                    waitg(1)
                    scale(b0, 0)
                    pltpu.sync_copy(rows.at[0], spmem.at[dref.at[b0]], add=True)

                    @pl.when(b0 + 2 < NB3)
                    def _g0():
                        issueg(b0 + 2, 0)

                    scale(b0 + 1, 1)
                    pltpu.sync_copy(rows.at[1], spmem.at[dref.at[b0 + 1]], add=True)

                    @pl.when(b0 + 3 < NB3)
                    def _g1():
                        issueg(b0 + 3, 1)

                    return 0

                lax.fori_loop(0, NB3 // 2, super2, 0)
                plsc.subcore_barrier()
                pltpu.sync_copy(
                    spmem.at[pl.ds(sub * NSLICE, NSLICE)],
                    sp_hbm.at[core, c, pl.ds(r * RSPAN + sub * NSLICE, NSLICE)])
                plsc.subcore_barrier()
                return 0

            lax.fori_loop(0, NCH, cbody, 0)
            return 0

        lax.fori_loop(0, NR, rbody, 0)

    return sc_sp


_sc_alpha = _make_sc_alpha()
_sc_softmax = _make_sc_softmax()
_sc_combine = _make_sc_combine()
_sc_sp = _make_sc_sp()


def kernel(x, edge_index, edge_attr, mW0, mb0, mW1, mb1, mW2, mb2, gWl, gbl, gWr, gbr, gWe, gatt, gbo, lW, lb, cW, cb, fW1, fb1, fW2, fb2):
    src = edge_index[0].astype(jnp.int32)
    dst = edge_index[1].astype(jnp.int32)
    src_p = jnp.pad(src, (0, EPAD - E))
    dst_p = jnp.pad(dst, (0, EPAD - E))
    ea_p = jnp.pad(edge_attr[:, 0], (0, EPAD - E))
    dst2 = dst_p.reshape(NW, NB3, B3)

    h = _mlp(x, mW0, mb0, mW1, mb1, mW2, mb2)
    for i in range(L):
        xl4, xr4, li = _pre(h, gWl[i], gbl[i], gWr[i], gbr[i], lW[i], lb[i])
        gwe = gWe[i].reshape(D)
        gat = gatt[i]
        alpha, wmax = _sc_alpha(xl4, xr4, src_p, dst_p, ea_p, gwe, gat)
        ae, asums = _sc_softmax(alpha, wmax, dst_p)
        asum = _sc_combine(asums)
        sp2 = _sc_sp(xl4, src_p, dst2, ae, asum)[:, :, :N, :]
        cb_eff = cb[i] + gbo[i] @ cW[i][:D]
        h = _post(sp2, li, h, cW[i], cb_eff, fW1[i], fb1[i], fW2[i], fb2[i])
    return h


# speedup vs baseline: 1.3520x; 1.3520x over previous
import functools

import jax
import jax.numpy as jnp
from jax import lax
from jax.experimental import pallas as pl
from jax.experimental.pallas import tpu as pltpu
from jax.experimental.pallas import tpu_sc as plsc

N = 10000
E = 160000
DIN = 256
D = 512
FF = 1024
L = 2

RB = 1000
NCH = 4
CW = 128

NW = 32
EPAD = 163840
EW = EPAD // NW
B1 = 32
NB1 = EW // B1
B3 = 128
NB3 = EW // B3
NPAD = 12288
NR = 2
RSPAN = 5120
NSPM = RSPAN + 8
NSLICE = RSPAN // 16

_MESH = plsc.VectorSubcoreMesh(core_axis_name="c", subcore_axis_name="s")
_SC_PARAMS = pltpu.CompilerParams(needs_layout_passes=False)


def _relu6(v):
    return jnp.clip(v, 0.0, 6.0)


def _full(*shape):
    return pl.BlockSpec(shape, lambda i: (0,) * len(shape))


def _mlp_body(x_ref, w0_ref, b0_ref, w1_ref, b1_ref, w2_ref, b2_ref, out_ref):
    h = _relu6(jnp.dot(x_ref[...], w0_ref[...], preferred_element_type=jnp.float32) + b0_ref[...])
    h = _relu6(jnp.dot(h, w1_ref[...], preferred_element_type=jnp.float32) + b1_ref[...])
    out_ref[...] = jnp.dot(h, w2_ref[...], preferred_element_type=jnp.float32) + b2_ref[...]


def _mlp(x, mW0, mb0, mW1, mb1, mW2, mb2):
    return pl.pallas_call(
        _mlp_body,
        grid=(N // RB,),
        in_specs=[
            pl.BlockSpec((RB, DIN), lambda i: (i, 0)),
            _full(DIN, 512), _full(1, 512), _full(512, 256), _full(1, 256),
            _full(256, D), _full(1, D),
        ],
        out_specs=pl.BlockSpec((RB, D), lambda i: (i, 0)),
        out_shape=jax.ShapeDtypeStruct((N, D), jnp.float32),
    )(x, mW0, mb0.reshape(1, 512), mW1, mb1.reshape(1, 256), mW2, mb2.reshape(1, D))


def _pre_body(h_ref, wl_ref, bl_ref, wr_ref, br_ref, wc_ref, bc_ref,
              xl4_ref, xr4_ref, li_ref):
    h = h_ref[...]
    xl = jnp.dot(h, wl_ref[...], preferred_element_type=jnp.float32) + bl_ref[...]
    xr = jnp.dot(h, wr_ref[...], preferred_element_type=jnp.float32) + br_ref[...]
    for c in range(NCH):
        xl4_ref[c] = xl[:, c * CW:(c + 1) * CW]
        xr4_ref[c] = xr[:, c * CW:(c + 1) * CW]
    li_ref[...] = jnp.dot(h, wc_ref[...], preferred_element_type=jnp.float32) + bc_ref[...]


def _pre(h, gWl, gbl, gWr, gbr, lW, lb):
    chunk4 = pl.BlockSpec((NCH, RB, CW), lambda i: (0, i, 0))
    return pl.pallas_call(
        _pre_body,
        grid=(N // RB,),
        in_specs=[
            pl.BlockSpec((RB, D), lambda i: (i, 0)),
            _full(D, D), _full(1, D), _full(D, D), _full(1, D), _full(D, D), _full(1, D),
        ],
        out_specs=[chunk4, chunk4, pl.BlockSpec((RB, D), lambda i: (i, 0))],
        out_shape=[jax.ShapeDtypeStruct((NCH, N, CW), jnp.float32)] * 2
        + [jax.ShapeDtypeStruct((N, D), jnp.float32)],
    )(h, gWl, gbl.reshape(1, D), gWr, gbr.reshape(1, D), lW, lb.reshape(1, D))


def _post_body(sp2_ref, li_ref, h_ref, cwt_ref, cwb_ref, cbe_ref,
               w1_ref, b1_ref, w2_ref, b2_ref, out_ref):
    acc = cbe_ref[...] + h_ref[...]
    for p in range(2):
        for c in range(NCH):
            acc = acc + jnp.dot(sp2_ref[p, c], cwt_ref[c], preferred_element_type=jnp.float32)
    fus = acc + jnp.dot(li_ref[...], cwb_ref[...], preferred_element_type=jnp.float32)
    ffn = _relu6(jnp.dot(fus, w1_ref[...], preferred_element_type=jnp.float32) + b1_ref[...])
    ffn = jnp.dot(ffn, w2_ref[...], preferred_element_type=jnp.float32) + b2_ref[...]
    out_ref[...] = _relu6(fus + ffn)


def _post(sp2, li, h, cW, cb_eff, fW1, fb1, fW2, fb2):
    cwt = cW[:D].reshape(NCH, CW, D)
    cwb = cW[D:]
    return pl.pallas_call(
        _post_body,
        grid=(N // RB,),
        in_specs=[
            pl.BlockSpec((2, NCH, RB, CW), lambda i: (0, 0, i, 0)),
            pl.BlockSpec((RB, D), lambda i: (i, 0)),
            pl.BlockSpec((RB, D), lambda i: (i, 0)),
            _full(NCH, CW, D), _full(D, D), _full(1, D),
            _full(D, FF), _full(1, FF), _full(FF, D), _full(1, D),
        ],
        out_specs=pl.BlockSpec((RB, D), lambda i: (i, 0)),
        out_shape=jax.ShapeDtypeStruct((N, D), jnp.float32),
    )(sp2, li, h, cwt, cwb, cb_eff.reshape(1, D),
      fW1, fb1.reshape(1, FF), fW2, fb2.reshape(1, D))


def _wid():
    return lax.axis_index("s") * 2 + lax.axis_index("c")


def _make_sc_alpha():
    @functools.partial(
        pl.kernel,
        out_type=[
            jax.ShapeDtypeStruct((EPAD,), jnp.float32),
            jax.ShapeDtypeStruct((NW * 128,), jnp.float32),
        ],
        mesh=_MESH,
        compiler_params=_SC_PARAMS,
        scratch_types=[
            pltpu.VMEM((EW,), jnp.int32),
            pltpu.VMEM((EW,), jnp.int32),
            pltpu.VMEM((EW,), jnp.float32),
            pltpu.VMEM((D,), jnp.float32),
            pltpu.VMEM((D,), jnp.float32),
            pltpu.VMEM((2, NCH, B1, CW), jnp.float32),
            pltpu.VMEM((2, NCH, B1, CW), jnp.float32),
            pltpu.VMEM((256,), jnp.float32),
            pltpu.VMEM((EW,), jnp.float32),
            pltpu.VMEM((128,), jnp.float32),
            pltpu.SemaphoreType.DMA,
            pltpu.SemaphoreType.DMA,
        ],
    )
    def sc_alpha(xl4, xr4,
                 src_hbm, dst_hbm, ea_hbm, gwe_hbm, gatt_hbm,
                 alpha_hbm, wmax_hbm,
                 src_v, dst_v, ea_v, gwe_v, gatt_v, bufl, bufr, partials,
                 alpha_v, wmax_v, sem0, sem1):
        xlc = tuple(xl4.at[c] for c in range(NCH))
        xrc = tuple(xr4.at[c] for c in range(NCH))
        w = _wid()
        base = w * EW
        pltpu.sync_copy(src_hbm.at[pl.ds(base, EW)], src_v)
        pltpu.sync_copy(dst_hbm.at[pl.ds(base, EW)], dst_v)
        pltpu.sync_copy(ea_hbm.at[pl.ds(base, EW)], ea_v)
        pltpu.sync_copy(gwe_hbm, gwe_v)
        pltpu.sync_copy(gatt_hbm, gatt_v)

        sems = (sem0, sem1)

        def issue(b, par):
            sidx = src_v.at[pl.ds(b * B1, B1)]
            didx = dst_v.at[pl.ds(b * B1, B1)]
            for c in range(NCH):
                pltpu.async_copy(xlc[c].at[sidx], bufl.at[par].at[c], sems[par])
                pltpu.async_copy(xrc[c].at[didx], bufr.at[par].at[c], sems[par])

        def wait_all(par):
            for c in range(NCH):
                pltpu.make_async_copy(xlc[c].at[pl.ds(0, B1)],
                                      bufl.at[par].at[c], sems[par]).wait()
                pltpu.make_async_copy(xrc[c].at[pl.ds(0, B1)],
                                      bufr.at[par].at[c], sems[par]).wait()

        def compute(b, par, maxvec):
            def group(g, mv):
                ebase = b * B1 + g * 16
                ea16 = ea_v[pl.ds(ebase, 16)]
                eab = [ea16[j] for j in range(16)]
                acc = [jnp.zeros((16,), jnp.float32) for _ in range(16)]
                for k in range(D // 16):
                    c, kk = divmod(k, CW // 16)
                    gwe_k = gwe_v[pl.ds(k * 16, 16)]
                    gatt_k = gatt_v[pl.ds(k * 16, 16)]
                    for j in range(16):
                        t = (bufl[par, c, g * 16 + j, pl.ds(kk * 16, 16)]
                             + bufr[par, c, g * 16 + j, pl.ds(kk * 16, 16)]
                             + eab[j] * gwe_k)
                        t = jnp.maximum(t, 0.2 * t)
                        acc[j] = acc[j] + t * gatt_k
                for j in range(16):
                    partials[pl.ds(j * 16, 16)] = acc[j]
                iota16 = lax.iota(jnp.int32, 16) * 16
                tot = jnp.zeros((16,), jnp.float32)
                for t_ in range(16):
                    col = plsc.load_gather(partials, [iota16 + t_])
                    tot = tot + col
                alpha_v[pl.ds(ebase, 16)] = tot
                return jnp.maximum(mv, tot)

            return lax.fori_loop(0, B1 // 16, group, maxvec)

        issue(0, 0)
        issue(1, 1)

        def super2(s2, maxvec):
            b0 = s2 * 2
            wait_all(0)
            maxvec = compute(b0, 0, maxvec)

            @pl.when(b0 + 2 < NB1)
            def _i0():
                issue(b0 + 2, 0)

            wait_all(1)
            maxvec = compute(b0 + 1, 1, maxvec)

            @pl.when(b0 + 3 < NB1)
            def _i1():
                issue(b0 + 3, 1)

            return maxvec

        maxvec = lax.fori_loop(0, NB1 // 2, super2,
                               jnp.full((16,), -3.4e38, jnp.float32))
        wmax_v[pl.ds(0, 16)] = maxvec
        for u in range(1, 8):
            wmax_v[pl.ds(u * 16, 16)] = jnp.full((16,), -3.4e38, jnp.float32)
        pltpu.sync_copy(alpha_v, alpha_hbm.at[pl.ds(base, EW)])
        pltpu.sync_copy(wmax_v, wmax_hbm.at[pl.ds(w * 128, 128)])

    return sc_alpha


def _make_sc_softmax():
    @functools.partial(
        pl.kernel,
        out_type=[
            jax.ShapeDtypeStruct((EPAD,), jnp.float32),
            jax.ShapeDtypeStruct((NW, NPAD), jnp.float32),
        ],
        mesh=_MESH,
        compiler_params=_SC_PARAMS,
        scratch_types=[
            pltpu.VMEM((EW,), jnp.float32),
            pltpu.VMEM((EW,), jnp.int32),
            pltpu.VMEM((EW,), jnp.float32),
            pltpu.VMEM((NW * 128,), jnp.float32),
            pltpu.VMEM((NPAD,), jnp.float32),
        ],
    )
    def sc_softmax(alpha_hbm, wmax_hbm, dst_hbm, ae_hbm, asums_hbm,
                   alpha_v, dst_v, ae_v, wmax_v, asum_v):
        w = _wid()
        base = w * EW
        pltpu.sync_copy(alpha_hbm.at[pl.ds(base, EW)], alpha_v)
        pltpu.sync_copy(dst_hbm.at[pl.ds(base, EW)], dst_v)
        pltpu.sync_copy(wmax_hbm, wmax_v)
        gmax = wmax_v[pl.ds(0, 16)]
        for k in range(1, NW * 8):
            gmax = jnp.maximum(gmax, wmax_v[pl.ds(k * 16, 16)])
        cmax = jnp.max(gmax)

        def zero(i, _):
            for u in range(8):
                asum_v[pl.ds(i * 128 + u * 16, 16)] = jnp.zeros((16,), jnp.float32)
            return 0

        lax.fori_loop(0, NPAD // 128, zero, 0)
        iota = lax.iota(jnp.int32, 16)

        def group(g, _):
            ebase = g * 16
            alpha16 = alpha_v[pl.ds(ebase, 16)]
            ae16 = jnp.exp(alpha16 - cmax)
            valid = (base + ebase + iota) < E
            ae16 = jnp.where(valid, ae16, 0.0)
            ae_v[pl.ds(ebase, 16)] = ae16
            dst16 = dst_v[pl.ds(ebase, 16)]
            plsc.addupdate_scatter(asum_v, [dst16], ae16)
            return 0

        lax.fori_loop(0, EW // 16, group, 0)
        pltpu.sync_copy(ae_v, ae_hbm.at[pl.ds(base, EW)])
        pltpu.sync_copy(asum_v, asums_hbm.at[w])

    return sc_softmax


def _make_sc_combine():
    @functools.partial(
        pl.kernel,
        out_type=jax.ShapeDtypeStruct((NPAD,), jnp.float32),
        mesh=_MESH,
        compiler_params=_SC_PARAMS,
        scratch_types=[
            pltpu.VMEM((384,), jnp.float32),
            pltpu.VMEM((384,), jnp.float32),
        ],
    )
    def sc_combine(asums_hbm, asum_hbm, acc_v, tmp_v):
        w = _wid()
        for t in range(24):
            acc_v[pl.ds(t * 16, 16)] = jnp.zeros((16,), jnp.float32)
        for k in range(NW):
            pltpu.sync_copy(asums_hbm.at[k].at[pl.ds(w * 384, 384)], tmp_v)
            for t in range(24):
                sl = pl.ds(t * 16, 16)
                acc_v[sl] = acc_v[sl] + tmp_v[sl]
        pltpu.sync_copy(acc_v, asum_hbm.at[pl.ds(w * 384, 384)])

    return sc_combine


def _make_sc_sp():
    @functools.partial(
        pl.kernel,
        out_type=jax.ShapeDtypeStruct((2, NCH, NR * RSPAN, CW), jnp.float32),
        mesh=_MESH,
        compiler_params=_SC_PARAMS,
        scratch_types=[
            pltpu.VMEM((EW,), jnp.int32),
            pltpu.VMEM((NB3, B3), jnp.int32),
            pltpu.VMEM((NR, NB3, B3), jnp.int32),
            pltpu.VMEM((EW,), jnp.float32),
            pltpu.VMEM((NPAD,), jnp.float32),
            pltpu.VMEM((EW,), jnp.float32),
            pltpu.VMEM((2, B3, CW), jnp.float32),
            pltpu.VMEM((64, CW), jnp.float32),
            pltpu.VMEM_SHARED((NSPM, CW), jnp.float32),
            pltpu.SemaphoreType.DMA,
        ],
    )
    def sc_sp(xl4,
              src_hbm, dst2_hbm, ae_hbm, asum_hbm, sp_hbm,
              src_v, dst2_v, dstR_v, ae_v, asum_v, a_v, rows, zbuf,
              spmem, semg):
        core = lax.axis_index("c")
        sub = lax.axis_index("s")
        w = sub * 2 + core
        base = w * EW
        pltpu.sync_copy(src_hbm.at[pl.ds(base, EW)], src_v)
        pltpu.sync_copy(dst2_hbm.at[w], dst2_v)
        pltpu.sync_copy(ae_hbm.at[pl.ds(base, EW)], ae_v)
        pltpu.sync_copy(asum_hbm, asum_v)

        def agroup(g, _):
            ebase = g * 16
            ae16 = ae_v[pl.ds(ebase, 16)]
            r_ = g // 8
            cc = g % 8
            dst16 = dst2_v[r_, pl.ds(cc * 16, 16)]
            as16 = plsc.load_gather(asum_v, [dst16])
            a_v[pl.ds(ebase, 16)] = ae16 / (as16 + 1e-16)
            trash = jnp.full((16,), RSPAN, jnp.int32)
            for rr in range(NR):
                dd = dst16 - rr * RSPAN
                ok = jnp.logical_and(dd >= 0, dd < RSPAN)
                dstR_v[rr, r_, pl.ds(cc * 16, 16)] = jnp.where(ok, dd, trash)
            return 0

        lax.fori_loop(0, EW // 16, agroup, 0)

        def zrow(i, _):
            for u in range(CW // 16):
                zbuf[i, pl.ds(u * 16, 16)] = jnp.zeros((16,), jnp.float32)
            return 0

        lax.fori_loop(0, 64, zrow, 0)

        def rbody(r, _r):
            dref = dstR_v.at[r]

            def cbody(c, _c):
                for z in range(NSLICE // 64):
                    pltpu.sync_copy(
                        zbuf, spmem.at[pl.ds(sub * NSLICE + z * 64, 64)])

                @pl.when(sub == 0)
                def _zero_trash():
                    pltpu.sync_copy(zbuf, spmem.at[pl.ds(RSPAN - 56, 64)])
                plsc.subcore_barrier()

                def issueg(b, par):
                    sidx = src_v.at[pl.ds(b * B3, B3)]
                    pltpu.async_copy(xl4.at[c].at[sidx], rows.at[par], semg)

                def waitg(par):
                    pltpu.make_async_copy(xl4.at[c].at[pl.ds(0, B3)],
                                          rows.at[par], semg).wait()

                def scale(b, par):
                    def sgroup(g, _g):
                        ebase = b * B3 + g * 16
                        a16 = a_v[pl.ds(ebase, 16)]
                        for j in range(16):
                            aj = a16[j]
                            for k in range(CW // 16):
                                sl = pl.ds(k * 16, 16)
                                rows[par, g * 16 + j, sl] = rows[par, g * 16 + j, sl] * aj
                        return 0

                    lax.fori_loop(0, B3 // 16, sgroup, 0)

                issueg(0, 0)
                issueg(1, 1)

                def super2(s2, _):
                    b0 = s2 * 2
                    waitg(0)
                    waitg(1)
                    scale(b0, 0)
                    pltpu.sync_copy(rows.at[0], spmem.at[dref.at[b0]], add=True)

                    @pl.when(b0 + 2 < NB3)
                    def _g0():
                        issueg(b0 + 2, 0)

                    scale(b0 + 1, 1)
                    pltpu.sync_copy(rows.at[1], spmem.at[dref.at[b0 + 1]], add=True)

                    @pl.when(b0 + 3 < NB3)
                    def _g1():
                        issueg(b0 + 3, 1)

                    return 0

                lax.fori_loop(0, NB3 // 2, super2, 0)
                plsc.subcore_barrier()
                pltpu.sync_copy(
                    spmem.at[pl.ds(sub * NSLICE, NSLICE)],
                    sp_hbm.at[core, c, pl.ds(r * RSPAN + sub * NSLICE, NSLICE)])
                plsc.subcore_barrier()
                return 0

            lax.fori_loop(0, NCH, cbody, 0)
            return 0

        lax.fori_loop(0, NR, rbody, 0)

    return sc_sp


_sc_alpha = _make_sc_alpha()
_sc_softmax = _make_sc_softmax()
_sc_combine = _make_sc_combine()
_sc_sp = _make_sc_sp()


def kernel(x, edge_index, edge_attr, mW0, mb0, mW1, mb1, mW2, mb2, gWl, gbl, gWr, gbr, gWe, gatt, gbo, lW, lb, cW, cb, fW1, fb1, fW2, fb2):
    src = edge_index[0].astype(jnp.int32)
    dst = edge_index[1].astype(jnp.int32)
    src_p = jnp.pad(src, (0, EPAD - E))
    dst_p = jnp.pad(dst, (0, EPAD - E))
    ea_p = jnp.pad(edge_attr[:, 0], (0, EPAD - E))
    dst2 = dst_p.reshape(NW, NB3, B3)

    h = _mlp(x, mW0, mb0, mW1, mb1, mW2, mb2)
    for i in range(L):
        xl4, xr4, li = _pre(h, gWl[i], gbl[i], gWr[i], gbr[i], lW[i], lb[i])
        gwe = gWe[i].reshape(D)
        gat = gatt[i]
        alpha, wmax = _sc_alpha(xl4, xr4, src_p, dst_p, ea_p, gwe, gat)
        ae, asums = _sc_softmax(alpha, wmax, dst_p)
        asum = _sc_combine(asums)
        sp2 = _sc_sp(xl4, src_p, dst2, ae, asum)[:, :, :N, :]
        cb_eff = cb[i] + gbo[i] @ cW[i][:D]
        h = _post(sp2, li, h, cW[i], cb_eff, fW1[i], fb1[i], fW2[i], fb2[i])
    return h
